# Initial kernel scaffold; baseline (speedup 1.0000x reference)
import jax
import jax.numpy as jnp
from jax.experimental import pallas as pl


def kernel(xs, label, center):
    raise NotImplementedError("write your pallas kernel here")


# speedup vs baseline: 1.9952x; 1.9952x over previous
import functools

import jax
import jax.numpy as jnp
from jax import lax
from jax.experimental import pallas as pl
from jax.experimental.pallas import tpu as pltpu
from jax.experimental.pallas import tpu_sc as plsc

N = 16384
D = 128
CLS = 10
CPAD = 16
W = 144
NC = 2
NS = 16
NW = NC * NS
ROWS_PER = N // NW
CHUNK = 128
NCHUNK = ROWS_PER // CHUNK


def _sc_partials(xs, labels):
    mesh = plsc.VectorSubcoreMesh(core_axis_name="c", subcore_axis_name="s")

    @functools.partial(
        pl.kernel,
        out_type=jax.ShapeDtypeStruct((NW, CPAD, W), jnp.float32),
        mesh=mesh,
        compiler_params=pltpu.CompilerParams(needs_layout_passes=False),
        scratch_types=[
            pltpu.VMEM((CHUNK, D), jnp.float32),
            pltpu.VMEM((CPAD, W), jnp.float32),
            pltpu.VMEM((ROWS_PER,), jnp.int32),
        ],
    )
    def body(xs_hbm, lbl_hbm, out_hbm, inbuf, tbl, lbl1d):
        cid = lax.axis_index("c")
        sid = lax.axis_index("s")
        wid = cid * NS + sid
        base = wid * ROWS_PER

        lane = lax.iota(jnp.int32, 16)
        zeros = jnp.zeros((16,), jnp.float32)

        for r in range(CPAD):
            for j in range(W // 16):
                tbl[r, pl.ds(16 * j, 16)] = zeros

        pltpu.sync_copy(lbl_hbm.at[pl.ds(base, ROWS_PER)], lbl1d)

        col = [lane + 16 * j for j in range(W // 16)]

        for g in range(NCHUNK):
            pltpu.sync_copy(xs_hbm.at[pl.ds(base + g * CHUNK, CHUNK)], inbuf)

            def row_body(i, carry, g=g):
                lblv = plsc.load_gather(
                    lbl1d, [jnp.full((16,), g * CHUNK, jnp.int32) + i])
                v = [inbuf[i, pl.ds(16 * j, 16)] for j in range(8)]
                sq01 = v[0] * v[0] + v[1] * v[1]
                sq23 = v[2] * v[2] + v[3] * v[3]
                sq45 = v[4] * v[4] + v[5] * v[5]
                sq67 = v[6] * v[6] + v[7] * v[7]
                sq = (sq01 + sq23) + (sq45 + sq67)
                s = jnp.sum(sq)
                sv = jnp.full((16,), s)
                ib = lax.bitcast_convert_type(sv, jnp.int32)
                y = lax.bitcast_convert_type(
                    jnp.int32(0x5F3759DF) - (ib >> 1), jnp.float32)
                h = sv * jnp.float32(-0.5)
                y = y * (jnp.float32(1.5) + h * y * y)
                y = y * (jnp.float32(1.5) + h * y * y)
                y = jnp.minimum(y, jnp.float32(1e12))
                nsqv = sv * y * y
                aux = jnp.where(lane == 0, nsqv,
                                jnp.where(lane == 1, jnp.float32(1.0),
                                          jnp.float32(0.0)))
                for j in range(8):
                    plsc.addupdate_scatter(tbl, [lblv, col[j]], v[j] * y)
                plsc.addupdate_scatter(tbl, [lblv, col[8]], aux)
                return carry

            lax.fori_loop(0, CHUNK, row_body, 0)

        pltpu.sync_copy(tbl, out_hbm.at[wid])

    return body(xs, labels)


def _combine(part, center):
    def body(part_ref, center_ref, out_ref):
        p = jnp.sum(part_ref[...], axis=0)
        c = center_ref[...]
        S = p[:CLS, :D]
        dot = jnp.sum(S * c, axis=1, keepdims=True)
        cnsq = jnp.sum(c * c, axis=1, keepdims=True)
        A = p[:CLS, D:D + 1]
        cnt = p[:CLS, D + 1:D + 2]
        per = jnp.where(cnt > 0,
                        (A - 2.0 * dot) / jnp.maximum(cnt, 1.0) + cnsq,
                        0.0)
        out_ref[...] = jnp.sum(per).reshape(1, 1)

    return pl.pallas_call(
        body,
        out_shape=jax.ShapeDtypeStruct((1, 1), jnp.float32),
    )(part, center)


def kernel(xs, label, center):
    labels = label.astype(jnp.int32)
    part = _sc_partials(xs, labels)
    out = _combine(part, center)
    return out[0, 0]
